# baseline (speedup 1.0000x reference)
import jax
import jax.numpy as jnp
from jax import lax
from jax.experimental import pallas as pl
from jax.experimental.pallas import tpu as pltpu
from jax.experimental.pallas import tpu_sc as plsc

_BIG = 8.0

OUTF = 64
INF = 1024
NROW = INF // 128
N_SC = 512
SUBT = 8


def _routing_body(etc_ref, g_ref, a0_ref, a1_ref, a0f_ref, a1f_ref,
                  ctc_ref, cv_ref):
    v0 = etc_ref[0] + g_ref[0]
    v1 = etc_ref[1] + g_ref[1]
    v2 = etc_ref[2] + g_ref[2]
    m0 = (v0 >= v1) & (v0 >= v2)
    m1 = jnp.logical_not(m0) & (v1 >= v2)
    rows = jax.lax.broadcasted_iota(jnp.int32, (OUTF, INF), 0)
    even = rows % 2 == 0
    a0_ref[...] = jnp.where(m0, 0.0, _BIG).astype(jnp.bfloat16)
    a1_ref[...] = jnp.where(m1, 0.0, _BIG).astype(jnp.bfloat16)
    sgn = jnp.where(even, _BIG, -_BIG)
    a0f_ref[...] = jnp.where(m0, 0.0, sgn)
    a1f_ref[...] = jnp.where(m1, 0.0, sgn)
    any_edge = jnp.any(m0 | m1, axis=1)
    evenc = jax.lax.broadcasted_iota(jnp.int32, (OUTF, 1), 0)[:, 0] % 2 == 0
    ctc_ref[...] = jnp.where(any_edge, _BIG,
                             jnp.where(evenc, 2.0, 1.0))[None, :]
    cinit = jnp.where(evenc,
                      jnp.where(any_edge, _BIG, 2.0),
                      jnp.where(any_edge, -_BIG, -1.0))
    cv_ref[...] = jnp.broadcast_to(cinit[:, None], cv_ref.shape)


def _routing(etc_t, g_t):
    return pl.pallas_call(
        _routing_body,
        out_shape=[
            jax.ShapeDtypeStruct((OUTF, INF), jnp.bfloat16),
            jax.ShapeDtypeStruct((OUTF, INF), jnp.bfloat16),
            jax.ShapeDtypeStruct((OUTF, INF), jnp.float32),
            jax.ShapeDtypeStruct((OUTF, INF), jnp.float32),
            jax.ShapeDtypeStruct((1, OUTF), jnp.float32),
            jax.ShapeDtypeStruct((OUTF, 32), jnp.float32),
        ],
    )(etc_t, g_t)


def _tc_body(a0_ref, a1_ref, c_ref, x_ref, out_ref):
    bn = x_ref.shape[0]
    xb = x_ref[...]
    p = xb.astype(jnp.bfloat16)
    q = (1.0 - xb).astype(jnp.bfloat16)
    pn = -p
    qn = -q
    mins = []
    for node in range(OUTF):
        pa, qa = (p, q) if node % 2 == 0 else (pn, qn)
        t = jnp.minimum(pa + a0_ref[node, :][None, :],
                        qa + a1_ref[node, :][None, :])
        mins.append(jnp.min(t, axis=1))
    m = jnp.stack(mins, axis=1).astype(jnp.float32)
    m = jnp.minimum(m, c_ref[...])
    cols = jax.lax.broadcasted_iota(jnp.int32, (bn, OUTF), 1)
    sgn = jnp.where(cols % 2 == 0, 1.0, -1.0)
    out_ref[...] = m * sgn


def _tc_main(a0, a1, ctc, x_tc):
    n_tc = x_tc.shape[0]
    bn = 512
    grid = (n_tc // bn,)
    return pl.pallas_call(
        _tc_body,
        grid=grid,
        in_specs=[
            pl.BlockSpec((OUTF, INF), lambda i: (0, 0)),
            pl.BlockSpec((OUTF, INF), lambda i: (0, 0)),
            pl.BlockSpec((1, OUTF), lambda i: (0, 0)),
            pl.BlockSpec((bn, INF), lambda i: (i, 0)),
        ],
        out_specs=pl.BlockSpec((bn, OUTF), lambda i: (i, 0)),
        out_shape=jax.ShapeDtypeStruct((n_tc, OUTF), jnp.float32),
    )(a0, a1, ctc, x_tc)


def _sc_body(x_hbm, a0_hbm, a1_hbm, c_hbm, out_hbm,
             xs, a0v, a1v, cvv, outb, sem):
    wid = lax.axis_index("c") * 16 + lax.axis_index("s")
    ne = 2 * wid
    pltpu.sync_copy(a0_hbm.at[pl.ds(ne * NROW, 2 * NROW)], a0v)
    pltpu.sync_copy(a1_hbm.at[pl.ds(ne * NROW, 2 * NROW)], a1v)
    pltpu.sync_copy(c_hbm.at[pl.ds(ne, 2)], cvv)

    inite = cvv[0, pl.ds(0, 16)]
    inito = cvv[1, pl.ds(0, 16)]

    def tile_body(t, _):
        pltpu.sync_copy(x_hbm.at[pl.ds(t * SUBT * NROW, SUBT * NROW)], xs)

        acc_e = [inite] * SUBT
        acc_o = [inito] * SUBT

        def jr_body(jr, carry):
            ae, ao = carry
            ae = list(ae)
            ao = list(ao)
            for h in range(8):
                off = h * 16
                a0e = a0v[jr, pl.ds(off, 16)]
                a1e = a1v[jr, pl.ds(off, 16)]
                a0o = a0v[NROW + jr, pl.ds(off, 16)]
                a1o = a1v[NROW + jr, pl.ds(off, 16)]
                for s in range(SUBT):
                    xv = xs[s * NROW + jr, pl.ds(off, 16)]
                    qv = 1.0 - xv
                    ae[s] = jnp.minimum(
                        ae[s], jnp.minimum(xv + a0e, qv + a1e))
                    ao[s] = jnp.maximum(
                        ao[s], jnp.maximum(xv + a0o, qv + a1o))
            return (tuple(ae), tuple(ao))

        acc_e, acc_o = lax.fori_loop(
            0, NROW, jr_body, (tuple(acc_e), tuple(acc_o)))

        for s in range(SUBT):
            outb[t * SUBT + s, pl.ds(0, 16)] = acc_e[s]
            outb[t * SUBT + s, pl.ds(16, 16)] = acc_o[s]
        return 0

    lax.fori_loop(0, N_SC // SUBT, tile_body, 0)

    pltpu.sync_copy(outb, out_hbm.at[pl.ds(wid * N_SC, N_SC)])


def _sc_slice(x_sc, a0f, a1f, cvf):
    mesh = plsc.VectorSubcoreMesh(core_axis_name="c", subcore_axis_name="s")
    fn = pl.kernel(
        _sc_body,
        out_type=jax.ShapeDtypeStruct((32 * N_SC, 32), jnp.float32),
        mesh=mesh,
        scratch_types=[
            pltpu.VMEM((SUBT * NROW, 128), jnp.float32),
            pltpu.VMEM((2 * NROW, 128), jnp.float32),
            pltpu.VMEM((2 * NROW, 128), jnp.float32),
            pltpu.VMEM((2, 32), jnp.float32),
            pltpu.VMEM((N_SC, 32), jnp.float32),
            pltpu.SemaphoreType.DMA,
        ],
    )
    return fn(x_sc.reshape(N_SC * NROW, 128),
              a0f.reshape(OUTF * NROW, 128),
              a1f.reshape(OUTF * NROW, 128), cvf)


def _fin_body(p_ref, out_ref):
    v = p_ref[...]
    out_ref[:, 0:1] = jnp.min(v[:, 0:16], axis=1, keepdims=True)
    out_ref[:, 1:2] = jnp.max(v[:, 16:32], axis=1, keepdims=True)


def _sc_finish(partials):
    return pl.pallas_call(
        _fin_body,
        out_shape=jax.ShapeDtypeStruct((32 * N_SC, 2), jnp.float32),
    )(partials)


def kernel(x, edge_type_count):
    n = x.shape[0]
    u = jax.random.uniform(jax.random.key(42), edge_type_count.shape,
                           minval=1e-6, maxval=1.0 - 1e-6)
    g = -jnp.log(-jnp.log(u))
    etc_t = jnp.transpose(edge_type_count, (2, 0, 1))
    g_t = jnp.transpose(g, (2, 0, 1))

    a0, a1, a0f, a1f, ctc, cvf = _routing(etc_t, g_t)

    n_tc = n - N_SC
    out_tc = _tc_main(a0, a1, ctc, x[:n_tc])
    partials = _sc_slice(x[n_tc:], a0f, a1f, cvf)
    fin = _sc_finish(partials)
    out_sc = fin.reshape(32, N_SC, 2).transpose(1, 0, 2).reshape(N_SC, OUTF)
    return jnp.concatenate([out_tc, out_sc], axis=0)


# speedup vs baseline: 3.1203x; 3.1203x over previous
import jax
import jax.numpy as jnp
from jax import lax
from jax.experimental import pallas as pl
from jax.experimental.pallas import tpu as pltpu
from jax.experimental.pallas import tpu_sc as plsc

_BIG = 8.0

OUTF = 64
INF = 1024
NROW = INF // 128
N_SC = 128
SUBT = 8


def _routing_body(etc_ref, g_ref, a0_ref, a1_ref, a0f_ref, a1f_ref,
                  ctc_ref, cv_ref):
    v0 = etc_ref[0] + g_ref[0]
    v1 = etc_ref[1] + g_ref[1]
    v2 = etc_ref[2] + g_ref[2]
    m0 = (v0 >= v1) & (v0 >= v2)
    m1 = jnp.logical_not(m0) & (v1 >= v2)
    rows = jax.lax.broadcasted_iota(jnp.int32, (OUTF, INF), 0)
    even = rows % 2 == 0
    a0_ref[...] = jnp.where(m0, 0.0, _BIG).astype(jnp.bfloat16)
    a1_ref[...] = jnp.where(m1, 0.0, _BIG).astype(jnp.bfloat16)
    sgn = jnp.where(even, _BIG, -_BIG)
    a0f_ref[...] = jnp.where(m0, 0.0, sgn)
    a1f_ref[...] = jnp.where(m1, 0.0, sgn)
    any_edge = jnp.any(m0 | m1, axis=1)
    evenc = jax.lax.broadcasted_iota(jnp.int32, (OUTF, 1), 0)[:, 0] % 2 == 0
    ctc_ref[...] = jnp.where(any_edge, _BIG,
                             jnp.where(evenc, 2.0, 1.0))[None, :]
    cinit = jnp.where(evenc,
                      jnp.where(any_edge, _BIG, 2.0),
                      jnp.where(any_edge, -_BIG, -1.0))
    cv_ref[...] = jnp.broadcast_to(cinit[:, None], cv_ref.shape)


def _routing(etc_t, g_t):
    return pl.pallas_call(
        _routing_body,
        out_shape=[
            jax.ShapeDtypeStruct((OUTF, INF), jnp.bfloat16),
            jax.ShapeDtypeStruct((OUTF, INF), jnp.bfloat16),
            jax.ShapeDtypeStruct((OUTF, INF), jnp.float32),
            jax.ShapeDtypeStruct((OUTF, INF), jnp.float32),
            jax.ShapeDtypeStruct((1, OUTF), jnp.float32),
            jax.ShapeDtypeStruct((OUTF, 32), jnp.float32),
        ],
    )(etc_t, g_t)


def _tc_body(a0_ref, a1_ref, c_ref, x_ref, out_ref):
    bn = x_ref.shape[0]
    xb = x_ref[...]
    p = xb.astype(jnp.bfloat16)
    q = (1.0 - xb).astype(jnp.bfloat16)
    pn = -p
    qn = -q
    mins = []
    for node in range(OUTF):
        pa, qa = (p, q) if node % 2 == 0 else (pn, qn)
        t = jnp.minimum(pa + a0_ref[node, :][None, :],
                        qa + a1_ref[node, :][None, :])
        mins.append(jnp.min(t, axis=1))
    m = jnp.stack(mins, axis=1).astype(jnp.float32)
    m = jnp.minimum(m, c_ref[...])
    cols = jax.lax.broadcasted_iota(jnp.int32, (bn, OUTF), 1)
    sgn = jnp.where(cols % 2 == 0, 1.0, -1.0)
    out_ref[...] = m * sgn


def _tc_main(a0, a1, ctc, x_tc):
    n_tc = x_tc.shape[0]
    bn = 512
    grid = (n_tc // bn,)
    return pl.pallas_call(
        _tc_body,
        grid=grid,
        in_specs=[
            pl.BlockSpec((OUTF, INF), lambda i: (0, 0)),
            pl.BlockSpec((OUTF, INF), lambda i: (0, 0)),
            pl.BlockSpec((1, OUTF), lambda i: (0, 0)),
            pl.BlockSpec((bn, INF), lambda i: (i, 0)),
        ],
        out_specs=pl.BlockSpec((bn, OUTF), lambda i: (i, 0)),
        out_shape=jax.ShapeDtypeStruct((n_tc, OUTF), jnp.float32),
    )(a0, a1, ctc, x_tc)


def _sc_body(x_hbm, a0_hbm, a1_hbm, c_hbm, out_hbm,
             xs, a0v, a1v, cvv, outb, sem):
    wid = lax.axis_index("c") * 16 + lax.axis_index("s")
    ne = 2 * wid
    pltpu.sync_copy(a0_hbm.at[pl.ds(ne * NROW, 2 * NROW)], a0v)
    pltpu.sync_copy(a1_hbm.at[pl.ds(ne * NROW, 2 * NROW)], a1v)
    pltpu.sync_copy(c_hbm.at[pl.ds(ne, 2)], cvv)

    inite = cvv[0, pl.ds(0, 16)]
    inito = cvv[1, pl.ds(0, 16)]

    def tile_body(t, _):
        pltpu.sync_copy(x_hbm.at[pl.ds(t * SUBT * NROW, SUBT * NROW)], xs)

        acc_e = [inite] * SUBT
        acc_o = [inito] * SUBT

        def jr_body(jr, carry):
            ae, ao = carry
            ae = list(ae)
            ao = list(ao)
            for h in range(8):
                off = h * 16
                a0e = a0v[jr, pl.ds(off, 16)]
                a1e = a1v[jr, pl.ds(off, 16)]
                a0o = a0v[NROW + jr, pl.ds(off, 16)]
                a1o = a1v[NROW + jr, pl.ds(off, 16)]
                for s in range(SUBT):
                    xv = xs[s * NROW + jr, pl.ds(off, 16)]
                    qv = 1.0 - xv
                    ae[s] = jnp.minimum(
                        ae[s], jnp.minimum(xv + a0e, qv + a1e))
                    ao[s] = jnp.maximum(
                        ao[s], jnp.maximum(xv + a0o, qv + a1o))
            return (tuple(ae), tuple(ao))

        acc_e, acc_o = lax.fori_loop(
            0, NROW, jr_body, (tuple(acc_e), tuple(acc_o)))

        for s in range(SUBT):
            outb[t * SUBT + s, pl.ds(0, 16)] = acc_e[s]
            outb[t * SUBT + s, pl.ds(16, 16)] = acc_o[s]
        return 0

    lax.fori_loop(0, N_SC // SUBT, tile_body, 0)

    pltpu.sync_copy(outb, out_hbm.at[pl.ds(wid * N_SC, N_SC)])


def _sc_slice(x_sc, a0f, a1f, cvf):
    mesh = plsc.VectorSubcoreMesh(core_axis_name="c", subcore_axis_name="s")
    fn = pl.kernel(
        _sc_body,
        out_type=jax.ShapeDtypeStruct((32 * N_SC, 32), jnp.float32),
        mesh=mesh,
        scratch_types=[
            pltpu.VMEM((SUBT * NROW, 128), jnp.float32),
            pltpu.VMEM((2 * NROW, 128), jnp.float32),
            pltpu.VMEM((2 * NROW, 128), jnp.float32),
            pltpu.VMEM((2, 32), jnp.float32),
            pltpu.VMEM((N_SC, 32), jnp.float32),
            pltpu.SemaphoreType.DMA,
        ],
    )
    return fn(x_sc.reshape(N_SC * NROW, 128),
              a0f.reshape(OUTF * NROW, 128),
              a1f.reshape(OUTF * NROW, 128), cvf)


def _fin_body(p_ref, out_ref):
    v = p_ref[...]
    out_ref[:, 0:1] = jnp.min(v[:, 0:16], axis=1, keepdims=True)
    out_ref[:, 1:2] = jnp.max(v[:, 16:32], axis=1, keepdims=True)


def _sc_finish(partials):
    return pl.pallas_call(
        _fin_body,
        out_shape=jax.ShapeDtypeStruct((32 * N_SC, 2), jnp.float32),
    )(partials)


def kernel(x, edge_type_count):
    n = x.shape[0]
    u = jax.random.uniform(jax.random.key(42), edge_type_count.shape,
                           minval=1e-6, maxval=1.0 - 1e-6)
    g = -jnp.log(-jnp.log(u))
    etc_t = jnp.transpose(edge_type_count, (2, 0, 1))
    g_t = jnp.transpose(g, (2, 0, 1))

    a0, a1, a0f, a1f, ctc, cvf = _routing(etc_t, g_t)

    n_tc = n - N_SC
    out_tc = _tc_main(a0, a1, ctc, x[:n_tc])
    partials = _sc_slice(x[n_tc:], a0f, a1f, cvf)
    fin = _sc_finish(partials)
    out_sc = fin.reshape(32, N_SC, 2).transpose(1, 0, 2).reshape(N_SC, OUTF)
    return jnp.concatenate([out_tc, out_sc], axis=0)


# speedup vs baseline: 5.6798x; 1.8202x over previous
import jax
import jax.numpy as jnp
from jax.experimental import pallas as pl
from jax.experimental.pallas import tpu as pltpu

_BIG = 8.0


def _daa_body(etc_ref, g_ref, x_ref, out_ref, a0_ref, a1_ref, c_ref):
    out_feats, in_feats = a0_ref.shape
    bn = x_ref.shape[0]

    @pl.when(pl.program_id(0) == 0)
    def _compute_routing():
        v0 = etc_ref[0] + g_ref[0]
        v1 = etc_ref[1] + g_ref[1]
        v2 = etc_ref[2] + g_ref[2]
        m0 = (v0 >= v1) & (v0 >= v2)
        m1 = jnp.logical_not(m0) & (v1 >= v2)
        a0_ref[...] = jnp.where(m0, 0.0, _BIG).astype(jnp.bfloat16)
        a1_ref[...] = jnp.where(m1, 0.0, _BIG).astype(jnp.bfloat16)
        any_edge = jnp.any(m0 | m1, axis=1)
        rows1 = jax.lax.broadcasted_iota(jnp.int32, (out_feats, 1), 0)
        neutral = jnp.where(rows1 % 2 == 0, 2.0, 1.0)[:, 0]
        c_ref[...] = jnp.where(any_edge, _BIG, neutral)[None, :]

    xb = x_ref[...]
    p = xb.astype(jnp.bfloat16)
    q = (1.0 - xb).astype(jnp.bfloat16)
    pn = -p
    qn = -q
    mins = []
    for node in range(out_feats):
        pa, qa = (p, q) if node % 2 == 0 else (pn, qn)
        t = jnp.minimum(pa + a0_ref[node, :][None, :],
                        qa + a1_ref[node, :][None, :])
        mins.append(jnp.min(t, axis=1))
    m = jnp.stack(mins, axis=1).astype(jnp.float32)
    m = jnp.minimum(m, c_ref[...])
    cols = jax.lax.broadcasted_iota(jnp.int32, (bn, out_feats), 1)
    sgn = jnp.where(cols % 2 == 0, 1.0, -1.0)
    out_ref[...] = m * sgn


def kernel(x, edge_type_count):
    n, in_feats = x.shape
    out_feats = edge_type_count.shape[0]
    u = jax.random.uniform(jax.random.key(42), edge_type_count.shape,
                           minval=1e-6, maxval=1.0 - 1e-6)
    g = -jnp.log(-jnp.log(u))
    etc_t = jnp.transpose(edge_type_count, (2, 0, 1))
    g_t = jnp.transpose(g, (2, 0, 1))

    bn = 1024
    grid = (n // bn,)
    return pl.pallas_call(
        _daa_body,
        grid=grid,
        in_specs=[
            pl.BlockSpec((3, out_feats, in_feats), lambda i: (0, 0, 0)),
            pl.BlockSpec((3, out_feats, in_feats), lambda i: (0, 0, 0)),
            pl.BlockSpec((bn, in_feats), lambda i: (i, 0)),
        ],
        out_specs=pl.BlockSpec((bn, out_feats), lambda i: (i, 0)),
        out_shape=jax.ShapeDtypeStruct((n, out_feats), jnp.float32),
        scratch_shapes=[
            pltpu.VMEM((out_feats, in_feats), jnp.bfloat16),
            pltpu.VMEM((out_feats, in_feats), jnp.bfloat16),
            pltpu.VMEM((1, out_feats), jnp.float32),
        ],
    )(etc_t, g_t, x)
